# baseline (speedup 1.0000x reference)
import functools

import jax
import jax.numpy as jnp
import numpy as np
from jax import lax
from jax.experimental import pallas as pl
from jax.experimental.pallas import tpu as pltpu
from jax.experimental.pallas import tpu_sc as plsc

F32 = jnp.float32
NSC = 2
NSUB = 16
LANES = 16
CHUNK = 128
ROWW = 16
_TCB = 1792


def _idxmap(h, c):
    lane = np.arange(LANES)
    return np.where(lane < h, lane, np.clip((lane - h) // c, 0, h - 1)).astype(np.int32)


def _vdot(a, b):
    out = a[:, 0:1] * b[0:1, :]
    for i in range(1, a.shape[1]):
        out = out + a[:, i:i + 1] * b[i:i + 1, :]
    return out


def _alpha_heads(p, h_next, c_next):
    outs = []
    for hd in range(h_next):
        base = h_next + hd * c_next
        s = p[:, base:base + 1]
        for cc in range(1, c_next):
            s = s + p[:, base + cc:base + cc + 1]
        outs.append(s)
    return outs


def _proj_core(hm, avs_ref, avd_ref, h_next, c_next):
    lane = lax.broadcasted_iota(jnp.int32, hm.shape, 1)
    zerov = jnp.zeros_like(hm[:, 0:1])
    st = hm
    dt = jnp.zeros_like(hm)
    als = _alpha_heads(hm * avs_ref[...], h_next, c_next)
    ald = _alpha_heads(hm * avd_ref[...], h_next, c_next)
    for hd in range(h_next):
        st = st + jnp.where(lane == hd, als[hd], zerov)
        dt = dt + jnp.where(lane == hd, ald[hd], zerov)
    return st, dt


def _proj_body(h_next, c_next, x_ref, w_ref, avs_ref, avd_ref, st_ref, dt_ref):
    hm = jnp.dot(x_ref[...], w_ref[...], preferred_element_type=F32)
    st, dt = _proj_core(hm, avs_ref, avd_ref, h_next, c_next)
    st_ref[...] = st
    dt_ref[...] = dt


def _proj(x, w, avs, avd, h_next, c_next, n_pad):
    grid = n_pad // _TCB
    return pl.pallas_call(
        functools.partial(_proj_body, h_next, c_next),
        grid=(grid,),
        in_specs=[
            pl.BlockSpec((_TCB, x.shape[1]), lambda i: (i, 0)),
            pl.BlockSpec(w.shape, lambda i: (0, 0)),
            pl.BlockSpec((1, LANES), lambda i: (0, 0)),
            pl.BlockSpec((1, LANES), lambda i: (0, 0)),
        ],
        out_specs=[
            pl.BlockSpec((_TCB, ROWW), lambda i: (i, 0)),
            pl.BlockSpec((_TCB, ROWW), lambda i: (i, 0)),
        ],
        out_shape=[
            jax.ShapeDtypeStruct((n_pad, ROWW), F32),
            jax.ShapeDtypeStruct((n_pad, ROWW), F32),
        ],
    )(x, w, avs, avd)


def _finish_core(h, f, acc_ref, st_ref, dt_ref, rep_ref, bvec_ref):
    a = acc_ref[0] + acc_ref[1]
    s = st_ref[...]
    sl = s + dt_ref[...]
    ees = jnp.exp(jnp.maximum(sl, 0.2 * sl))
    lane = lax.broadcasted_iota(jnp.int32, s.shape, 1)
    eem = jnp.where(lane < h, ees, 0.0)
    rep = _vdot(eem, rep_ref[...])
    tot = a + rep * jnp.where(lane < h, 1.0, s)
    den = _vdot(jnp.where(lane < h, tot, 0.0), rep_ref[...])
    maskf = (lane >= h) & (lane < h + f)
    return jnp.maximum(jnp.where(maskf, tot / (den + 1e-16), 0.0)
                       + bvec_ref[...], 0.0)


def _finish_proj_body(h, f, h_next, c_next, acc_ref, st_ref, dt_ref,
                      rep_ref, bvec_ref, w_ref, avs_ref, avd_ref,
                      st2_ref, dt2_ref):
    hfeat = _finish_core(h, f, acc_ref, st_ref, dt_ref, rep_ref, bvec_ref)
    hm = jnp.dot(hfeat, w_ref[...], preferred_element_type=F32)
    st, dt = _proj_core(hm, avs_ref, avd_ref, h_next, c_next)
    st2_ref[...] = st
    dt2_ref[...] = dt


def _finish_proj(h, f, h_next, c_next, acc, st, dt, rep, bvec, w, avs,
                 avd, n_pad):
    grid = n_pad // _TCB
    blk = lambda i: (i, 0)
    return pl.pallas_call(
        functools.partial(_finish_proj_body, h, f, h_next, c_next),
        grid=(grid,),
        in_specs=[
            pl.BlockSpec((NSC, _TCB, ROWW), lambda i: (0, i, 0)),
            pl.BlockSpec((_TCB, ROWW), blk),
            pl.BlockSpec((_TCB, ROWW), blk),
            pl.BlockSpec((LANES, LANES), lambda i: (0, 0)),
            pl.BlockSpec((1, LANES), lambda i: (0, 0)),
            pl.BlockSpec((LANES, LANES), lambda i: (0, 0)),
            pl.BlockSpec((1, LANES), lambda i: (0, 0)),
            pl.BlockSpec((1, LANES), lambda i: (0, 0)),
        ],
        out_specs=[
            pl.BlockSpec((_TCB, ROWW), blk),
            pl.BlockSpec((_TCB, ROWW), blk),
        ],
        out_shape=[
            jax.ShapeDtypeStruct((n_pad, ROWW), F32),
            jax.ShapeDtypeStruct((n_pad, ROWW), F32),
        ],
    )(acc, st, dt, rep, bvec, w, avs, avd)


def _finish_only_body(h, f, acc_ref, st_ref, dt_ref, rep_ref, bvec_ref,
                      hf_ref):
    hf_ref[...] = _finish_core(h, f, acc_ref, st_ref, dt_ref, rep_ref,
                               bvec_ref)


def _finish_only(h, f, acc, st, dt, rep, bvec, n_pad):
    grid = n_pad // _TCB
    blk = lambda i: (i, 0)
    return pl.pallas_call(
        functools.partial(_finish_only_body, h, f),
        grid=(grid,),
        in_specs=[
            pl.BlockSpec((NSC, _TCB, ROWW), lambda i: (0, i, 0)),
            pl.BlockSpec((_TCB, ROWW), blk),
            pl.BlockSpec((_TCB, ROWW), blk),
            pl.BlockSpec((LANES, LANES), lambda i: (0, 0)),
            pl.BlockSpec((1, LANES), lambda i: (0, 0)),
        ],
        out_specs=pl.BlockSpec((_TCB, ROWW), blk),
        out_shape=jax.ShapeDtypeStruct((n_pad, ROWW), F32),
    )(acc, st, dt, rep, bvec)


def _mlp_body(h_ref, w1_ref, b1_ref, w2_ref, b2_ref, o_ref):
    z = jnp.maximum(
        jnp.dot(h_ref[...], w1_ref[...], preferred_element_type=F32)
        + b1_ref[...], 0.0)
    o_ref[...] = (jnp.dot(z, w2_ref[...], preferred_element_type=F32)
                  + b2_ref[...])


def _mlp(hh, w1, b1, w2, b2):
    m = hh.shape[0]
    mb = 1000
    blk = lambda i: (i, 0)
    return pl.pallas_call(
        _mlp_body,
        grid=(m // mb,),
        in_specs=[
            pl.BlockSpec((mb, hh.shape[1]), blk),
            pl.BlockSpec(w1.shape, lambda i: (0, 0)),
            pl.BlockSpec((1, w1.shape[1]), lambda i: (0, 0)),
            pl.BlockSpec(w2.shape, lambda i: (0, 0)),
            pl.BlockSpec((1, 1), lambda i: (0, 0)),
        ],
        out_specs=pl.BlockSpec((mb, 1), blk),
        out_shape=jax.ShapeDtypeStruct((m, 1), F32),
    )(hh, w1, b1.reshape(1, -1), w2, b2.reshape(1, -1))


def _make_sc_edge(h, c, n_pad, cw):
    assert cw % 8 == 0
    rows_per = n_pad // NSUB
    ng8 = cw // 8
    mesh = plsc.VectorSubcoreMesh(core_axis_name="c", subcore_axis_name="s")

    scratch = (
        [pltpu.VMEM((CHUNK,), jnp.int32) for _ in range(8)]
        + [pltpu.VMEM((CHUNK,), jnp.int32) for _ in range(8)]
        + [pltpu.VMEM((CHUNK,), jnp.int32) for _ in range(8)]
        + [pltpu.VMEM((CHUNK, ROWW), F32) for _ in range(4)]
        + [pltpu.VMEM((CHUNK, ROWW), F32) for _ in range(4)]
        + [pltpu.VMEM((CHUNK, ROWW), F32) for _ in range(4)]
        + [pltpu.VMEM_SHARED((n_pad, ROWW), F32)]
        + [pltpu.SemaphoreType.DMA for _ in range(16)]
    )

    @functools.partial(
        pl.kernel,
        mesh=mesh,
        compiler_params=pltpu.CompilerParams(use_tc_tiling_on_sc=False),
        out_type=jax.ShapeDtypeStruct((NSC, n_pad, ROWW), F32),
        scratch_types=scratch,
    )
    def sc_fn(st_hbm, dt_hbm, si_hbm, di_hbm, zer_hbm, acc_hbm, *scr):
        ixs = scr[0:8]
        ixg = scr[8:16]
        ixc = scr[16:24]
        sb = scr[24:28]
        db = scr[28:32]
        ob = scr[32:36]
        acc_sp = scr[36]
        sem_i = scr[37:45]
        sem_g = scr[45:49]
        sem_s = scr[49:53]

        cid = lax.axis_index("c")
        sid = lax.axis_index("s")
        wid = cid * NSUB + sid
        r0 = sid * rows_per
        pltpu.sync_copy(zer_hbm.at[pl.ds(r0, rows_per)],
                        acc_sp.at[pl.ds(r0, rows_per)])
        plsc.subcore_barrier()

        def idx_issue(cc, r):
            row = wid * cw + cc
            pltpu.async_copy(si_hbm.at[row], ixs[r], sem_i[r])
            pltpu.async_copy(di_hbm.at[row], ixg[r], sem_i[r])
            pltpu.async_copy(di_hbm.at[row], ixc[r], sem_i[r])

        def idx_wait(r):
            pltpu.make_async_copy(si_hbm.at[0], ixs[r], sem_i[r]).wait()
            pltpu.make_async_copy(di_hbm.at[0], ixg[r], sem_i[r]).wait()
            pltpu.make_async_copy(di_hbm.at[0], ixc[r], sem_i[r]).wait()

        def g_issue(r, q):
            pltpu.async_copy(st_hbm.at[ixs[r]], sb[q], sem_g[q])
            pltpu.async_copy(dt_hbm.at[ixg[r]], db[q], sem_g[q])

        def g_wait(r, q):
            pltpu.make_async_copy(st_hbm.at[ixs[r]], sb[q], sem_g[q]).wait()
            pltpu.make_async_copy(dt_hbm.at[ixg[r]], db[q], sem_g[q]).wait()

        def s_issue(r, q):
            pltpu.async_copy(ob[q], acc_sp.at[ixc[r]], sem_s[q], add=True)

        def s_wait(r, q):
            pltpu.make_async_copy(ob[q], acc_sp.at[ixc[r]], sem_s[q]).wait()

        lane = lax.iota(jnp.int32, LANES)
        m0 = (lane == 0) | ((lane >= h) & (lane < h + c))
        mh = lane < h

        def compute(q):
            sbuf, dbuf, obuf = sb[q], db[q], ob[q]

            def edge(e, carry):
                s_row = sbuf[e]
                t = s_row + dbuf[e]
                ee = jnp.exp(jnp.maximum(t, 0.2 * t))
                if h == 1:
                    g = ee[0]
                else:
                    g = jnp.where(m0, ee[0], ee[1])
                obuf[e] = g * jnp.where(mh, 1.0, s_row)
                return carry

            lax.fori_loop(0, CHUNK, edge, 0, unroll=8)

        for r in range(8):
            idx_issue(r, r)
        idx_wait(0)
        g_issue(0, 0)
        idx_wait(1)
        g_issue(1, 1)

        def group(gg, carry):
            for qq in range(8):
                q = qq % 4
                cc = gg * 8 + qq
                g_wait(qq, q)
                if qq < 4:
                    @pl.when(gg >= 1)
                    def _():
                        s_wait((qq + 4) % 8, q)
                        idx_issue(cc + 4, (qq + 4) % 8)
                else:
                    s_wait(qq - 4, q)

                    @pl.when(gg < ng8 - 1)
                    def _():
                        idx_issue(cc + 4, qq - 4)
                if qq < 6:
                    idx_wait(qq + 2)
                    g_issue(qq + 2, (qq + 2) % 4)
                else:
                    @pl.when(gg < ng8 - 1)
                    def _():
                        idx_wait(qq - 6)
                        g_issue(qq - 6, (qq + 2) % 4)
                compute(q)
                s_issue(qq, q)
            return carry

        lax.fori_loop(0, ng8, group, 0)
        for qq in range(4, 8):
            s_wait(qq, qq % 4)
        plsc.subcore_barrier()
        pltpu.sync_copy(acc_sp.at[pl.ds(r0, rows_per)],
                        acc_hbm.at[cid, pl.ds(r0, rows_per)])

    return sc_fn


def _pack_w_first(w, h_next):
    d_in, hc = w.shape
    return jnp.concatenate(
        [jnp.zeros((d_in, h_next), F32), w,
         jnp.zeros((d_in, ROWW - h_next - hc), F32)], axis=1)


def _pack_w_next(w, h_prev, h_next):
    wl = jnp.zeros((ROWW, ROWW), F32)
    return wl.at[h_prev:h_prev + w.shape[0],
                 h_next:h_next + w.shape[1]].set(w)


def _pack_avec(a, h_next):
    h, c = a.shape
    v = jnp.zeros((1, ROWW), F32)
    return v.at[0, h_next:h_next + h * c].set(a.reshape(-1))


def _repm(h, c):
    m = (_idxmap(h, c)[None, :] == np.arange(LANES)[:, None])
    return jnp.asarray(m.astype(np.float32))


def _bvec(h, b):
    v = jnp.zeros((1, ROWW), F32)
    return v.at[0, h:h + b.shape[0]].set(b)


def kernel(x, edge_index, W1, a_s1, a_d1, b1, W2, a_s2, a_d2, b2,
           W3, a_s3, a_d3, b3, Wl1, bl1, Wl2, bl2):
    n = x.shape[0]
    e = edge_index.shape[1]
    n_pad = ((n + 1 + _TCB - 1) // _TCB) * _TCB
    cw = -(-e // (NSC * NSUB * CHUNK))
    cw = ((cw + 7) // 8) * 8
    e_pad = NSC * NSUB * cw * CHUNK

    pad_e = e_pad - e
    src = jnp.concatenate(
        [edge_index[0], jnp.full((pad_e,), n, jnp.int32)]).reshape(-1, CHUNK)
    dst = jnp.concatenate(
        [edge_index[1], jnp.full((pad_e,), n, jnp.int32)]).reshape(-1, CHUNK)
    x_pad = jnp.pad(x, ((0, n_pad - n), (0, 0)))
    zer = jnp.zeros((n_pad, ROWW), F32)

    w1p = _pack_w_first(W1, 2)
    w2p = _pack_w_next(W2, 2, 2)
    w3p = _pack_w_next(W3, 2, 1)

    st1, dt1 = _proj(x_pad, w1p, _pack_avec(a_s1, 2), _pack_avec(a_d1, 2),
                     2, 5, n_pad)
    acc1 = _make_sc_edge(2, 5, n_pad, cw)(st1, dt1, src, dst, zer)
    st2, dt2 = _finish_proj(2, 10, 2, 2, acc1, st1, dt1, _repm(2, 5),
                            _bvec(2, b1), w2p,
                            _pack_avec(a_s2, 2), _pack_avec(a_d2, 2), n_pad)
    acc2 = _make_sc_edge(2, 2, n_pad, cw)(st2, dt2, src, dst, zer)
    st3, dt3 = _finish_proj(2, 4, 1, 2, acc2, st2, dt2, _repm(2, 2),
                            _bvec(2, b2), w3p,
                            _pack_avec(a_s3, 1), _pack_avec(a_d3, 1), n_pad)
    acc3 = _make_sc_edge(1, 2, n_pad, cw)(st3, dt3, src, dst, zer)
    hf3 = _finish_only(1, 2, acc3, st3, dt3, _repm(1, 2), _bvec(1, b3), n_pad)

    hh = hf3[:n, 1:3].reshape(n // 7, 14)
    out = _mlp(hh, Wl1, bl1, Wl2, bl2)
    return out[:, 0]


# speedup vs baseline: 1.0143x; 1.0143x over previous
import functools

import jax
import jax.numpy as jnp
import numpy as np
from jax import lax
from jax.experimental import pallas as pl
from jax.experimental.pallas import tpu as pltpu
from jax.experimental.pallas import tpu_sc as plsc

F32 = jnp.float32
NSC = 2
NSUB = 16
LANES = 16
CHUNK = 128
ROWW = 16
_TCB = 1792


def _idxmap(h, c):
    lane = np.arange(LANES)
    return np.where(lane < h, lane, np.clip((lane - h) // c, 0, h - 1)).astype(np.int32)


def _vdot(a, b):
    out = a[:, 0:1] * b[0:1, :]
    for i in range(1, a.shape[1]):
        out = out + a[:, i:i + 1] * b[i:i + 1, :]
    return out


def _alpha_heads(p, h_next, c_next):
    outs = []
    for hd in range(h_next):
        base = h_next + hd * c_next
        s = p[:, base:base + 1]
        for cc in range(1, c_next):
            s = s + p[:, base + cc:base + cc + 1]
        outs.append(s)
    return outs


def _proj_core(hm, avs_ref, avd_ref, h_next, c_next):
    lane = lax.broadcasted_iota(jnp.int32, hm.shape, 1)
    zerov = jnp.zeros_like(hm[:, 0:1])
    st = hm
    dt = jnp.zeros_like(hm)
    als = _alpha_heads(hm * avs_ref[...], h_next, c_next)
    ald = _alpha_heads(hm * avd_ref[...], h_next, c_next)
    for hd in range(h_next):
        st = st + jnp.where(lane == hd, als[hd], zerov)
        dt = dt + jnp.where(lane == hd, ald[hd], zerov)
    return st, dt


def _proj_body(h_next, c_next, x_ref, w_ref, avs_ref, avd_ref, st_ref, dt_ref):
    hm = jnp.dot(x_ref[...], w_ref[...], preferred_element_type=F32)
    st, dt = _proj_core(hm, avs_ref, avd_ref, h_next, c_next)
    st_ref[...] = st
    dt_ref[...] = dt


def _proj(x, w, avs, avd, h_next, c_next, n_pad):
    grid = n_pad // _TCB
    return pl.pallas_call(
        functools.partial(_proj_body, h_next, c_next),
        grid=(grid,),
        in_specs=[
            pl.BlockSpec((_TCB, x.shape[1]), lambda i: (i, 0)),
            pl.BlockSpec(w.shape, lambda i: (0, 0)),
            pl.BlockSpec((1, LANES), lambda i: (0, 0)),
            pl.BlockSpec((1, LANES), lambda i: (0, 0)),
        ],
        out_specs=[
            pl.BlockSpec((_TCB, ROWW), lambda i: (i, 0)),
            pl.BlockSpec((_TCB, ROWW), lambda i: (i, 0)),
        ],
        out_shape=[
            jax.ShapeDtypeStruct((n_pad, ROWW), F32),
            jax.ShapeDtypeStruct((n_pad, ROWW), F32),
        ],
    )(x, w, avs, avd)


def _finish_core(h, f, acc_ref, st_ref, dt_ref, rep_ref, bvec_ref):
    a = acc_ref[0] + acc_ref[1]
    s = st_ref[...]
    sl = s + dt_ref[...]
    ees = jnp.exp(jnp.maximum(sl, 0.2 * sl))
    lane = lax.broadcasted_iota(jnp.int32, s.shape, 1)
    eem = jnp.where(lane < h, ees, 0.0)
    rep = _vdot(eem, rep_ref[...])
    tot = a + rep * jnp.where(lane < h, 1.0, s)
    den = _vdot(jnp.where(lane < h, tot, 0.0), rep_ref[...])
    maskf = (lane >= h) & (lane < h + f)
    return jnp.maximum(jnp.where(maskf, tot / (den + 1e-16), 0.0)
                       + bvec_ref[...], 0.0)


def _finish_proj_body(h, f, h_next, c_next, acc_ref, st_ref, dt_ref,
                      rep_ref, bvec_ref, w_ref, avs_ref, avd_ref,
                      st2_ref, dt2_ref):
    hfeat = _finish_core(h, f, acc_ref, st_ref, dt_ref, rep_ref, bvec_ref)
    hm = jnp.dot(hfeat, w_ref[...], preferred_element_type=F32)
    st, dt = _proj_core(hm, avs_ref, avd_ref, h_next, c_next)
    st2_ref[...] = st
    dt2_ref[...] = dt


def _finish_proj(h, f, h_next, c_next, acc, st, dt, rep, bvec, w, avs,
                 avd, n_pad):
    grid = n_pad // _TCB
    blk = lambda i: (i, 0)
    return pl.pallas_call(
        functools.partial(_finish_proj_body, h, f, h_next, c_next),
        grid=(grid,),
        in_specs=[
            pl.BlockSpec((NSC, _TCB, ROWW), lambda i: (0, i, 0)),
            pl.BlockSpec((_TCB, ROWW), blk),
            pl.BlockSpec((_TCB, ROWW), blk),
            pl.BlockSpec((LANES, LANES), lambda i: (0, 0)),
            pl.BlockSpec((1, LANES), lambda i: (0, 0)),
            pl.BlockSpec((LANES, LANES), lambda i: (0, 0)),
            pl.BlockSpec((1, LANES), lambda i: (0, 0)),
            pl.BlockSpec((1, LANES), lambda i: (0, 0)),
        ],
        out_specs=[
            pl.BlockSpec((_TCB, ROWW), blk),
            pl.BlockSpec((_TCB, ROWW), blk),
        ],
        out_shape=[
            jax.ShapeDtypeStruct((n_pad, ROWW), F32),
            jax.ShapeDtypeStruct((n_pad, ROWW), F32),
        ],
    )(acc, st, dt, rep, bvec, w, avs, avd)


def _finish_only_body(h, f, acc_ref, st_ref, dt_ref, rep_ref, bvec_ref,
                      hf_ref):
    hf_ref[...] = _finish_core(h, f, acc_ref, st_ref, dt_ref, rep_ref,
                               bvec_ref)


def _finish_only(h, f, acc, st, dt, rep, bvec, n_pad):
    grid = n_pad // _TCB
    blk = lambda i: (i, 0)
    return pl.pallas_call(
        functools.partial(_finish_only_body, h, f),
        grid=(grid,),
        in_specs=[
            pl.BlockSpec((NSC, _TCB, ROWW), lambda i: (0, i, 0)),
            pl.BlockSpec((_TCB, ROWW), blk),
            pl.BlockSpec((_TCB, ROWW), blk),
            pl.BlockSpec((LANES, LANES), lambda i: (0, 0)),
            pl.BlockSpec((1, LANES), lambda i: (0, 0)),
        ],
        out_specs=pl.BlockSpec((_TCB, ROWW), blk),
        out_shape=jax.ShapeDtypeStruct((n_pad, ROWW), F32),
    )(acc, st, dt, rep, bvec)


def _mlp_body(h_ref, w1_ref, b1_ref, w2_ref, b2_ref, o_ref):
    z = jnp.maximum(
        jnp.dot(h_ref[...], w1_ref[...], preferred_element_type=F32)
        + b1_ref[...], 0.0)
    o_ref[...] = (jnp.dot(z, w2_ref[...], preferred_element_type=F32)
                  + b2_ref[...])


def _mlp(hh, w1, b1, w2, b2):
    m = hh.shape[0]
    mb = 1000
    blk = lambda i: (i, 0)
    return pl.pallas_call(
        _mlp_body,
        grid=(m // mb,),
        in_specs=[
            pl.BlockSpec((mb, hh.shape[1]), blk),
            pl.BlockSpec(w1.shape, lambda i: (0, 0)),
            pl.BlockSpec((1, w1.shape[1]), lambda i: (0, 0)),
            pl.BlockSpec(w2.shape, lambda i: (0, 0)),
            pl.BlockSpec((1, 1), lambda i: (0, 0)),
        ],
        out_specs=pl.BlockSpec((mb, 1), blk),
        out_shape=jax.ShapeDtypeStruct((m, 1), F32),
    )(hh, w1, b1.reshape(1, -1), w2, b2.reshape(1, -1))


def _make_sc_edge(h, c, n_pad, cw):
    assert cw % 8 == 0
    rows_per = n_pad // NSUB
    ng8 = cw // 8
    mesh = plsc.VectorSubcoreMesh(core_axis_name="c", subcore_axis_name="s")

    scratch = (
        [pltpu.VMEM((CHUNK,), jnp.int32) for _ in range(8)]
        + [pltpu.VMEM((CHUNK,), jnp.int32) for _ in range(8)]
        + [pltpu.VMEM((CHUNK, ROWW), F32) for _ in range(4)]
        + [pltpu.VMEM((CHUNK, ROWW), F32) for _ in range(4)]
        + [pltpu.VMEM((CHUNK, ROWW), F32) for _ in range(4)]
        + [pltpu.VMEM_SHARED((n_pad, ROWW), F32)]
        + [pltpu.SemaphoreType.DMA for _ in range(16)]
    )

    @functools.partial(
        pl.kernel,
        mesh=mesh,
        compiler_params=pltpu.CompilerParams(use_tc_tiling_on_sc=False),
        out_type=jax.ShapeDtypeStruct((NSC, n_pad, ROWW), F32),
        scratch_types=scratch,
    )
    def sc_fn(st_hbm, dt_hbm, si_hbm, di_hbm, zer_hbm, acc_hbm, *scr):
        ixs = scr[0:8]
        ixc = scr[8:16]
        sb = scr[16:20]
        db = scr[20:24]
        ob = scr[24:28]
        acc_sp = scr[28]
        sem_i = scr[29:37]
        sem_g = scr[37:41]
        sem_s = scr[41:45]

        cid = lax.axis_index("c")
        sid = lax.axis_index("s")
        wid = cid * NSUB + sid
        r0 = sid * rows_per
        pltpu.sync_copy(zer_hbm.at[pl.ds(r0, rows_per)],
                        acc_sp.at[pl.ds(r0, rows_per)])
        plsc.subcore_barrier()

        def idx_issue(cc, r):
            row = wid * cw + cc
            pltpu.async_copy(si_hbm.at[row], ixs[r], sem_i[r])
            pltpu.async_copy(di_hbm.at[row], ixc[r], sem_i[r])

        def idx_wait(r):
            pltpu.make_async_copy(si_hbm.at[0], ixs[r], sem_i[r]).wait()
            pltpu.make_async_copy(di_hbm.at[0], ixc[r], sem_i[r]).wait()

        def g_issue(r, q):
            pltpu.async_copy(st_hbm.at[ixs[r]], sb[q], sem_g[q])
            pltpu.async_copy(dt_hbm.at[ixc[r]], db[q], sem_g[q])

        def g_wait(r, q):
            pltpu.make_async_copy(st_hbm.at[ixs[r]], sb[q], sem_g[q]).wait()
            pltpu.make_async_copy(dt_hbm.at[ixc[r]], db[q], sem_g[q]).wait()

        def s_issue(r, q):
            pltpu.async_copy(ob[q], acc_sp.at[ixc[r]], sem_s[q], add=True)

        def s_wait(r, q):
            pltpu.make_async_copy(ob[q], acc_sp.at[ixc[r]], sem_s[q]).wait()

        lane = lax.iota(jnp.int32, LANES)
        m0 = (lane == 0) | ((lane >= h) & (lane < h + c))
        mh = lane < h

        def compute(q):
            sbuf, dbuf, obuf = sb[q], db[q], ob[q]

            def edge(e, carry):
                s_row = sbuf[e]
                t = s_row + dbuf[e]
                ee = jnp.exp(jnp.maximum(t, 0.2 * t))
                if h == 1:
                    g = ee[0]
                else:
                    g = jnp.where(m0, ee[0], ee[1])
                obuf[e] = g * jnp.where(mh, 1.0, s_row)
                return carry

            lax.fori_loop(0, CHUNK, edge, 0, unroll=4)

        for r in range(8):
            idx_issue(r, r)
        idx_wait(0)
        g_issue(0, 0)
        idx_wait(1)
        g_issue(1, 1)

        def group(gg, carry):
            for qq in range(8):
                q = qq % 4
                cc = gg * 8 + qq
                g_wait(qq, q)
                if qq < 4:
                    @pl.when(gg >= 1)
                    def _():
                        s_wait((qq + 4) % 8, q)
                        idx_issue(cc + 4, (qq + 4) % 8)
                else:
                    s_wait(qq - 4, q)

                    @pl.when(gg < ng8 - 1)
                    def _():
                        idx_issue(cc + 4, qq - 4)
                if qq < 6:
                    idx_wait(qq + 2)
                    g_issue(qq + 2, (qq + 2) % 4)
                else:
                    @pl.when(gg < ng8 - 1)
                    def _():
                        idx_wait(qq - 6)
                        g_issue(qq - 6, (qq + 2) % 4)
                compute(q)
                s_issue(qq, q)
            return carry

        lax.fori_loop(0, ng8, group, 0)
        for qq in range(4, 8):
            s_wait(qq, qq % 4)
        plsc.subcore_barrier()
        pltpu.sync_copy(acc_sp.at[pl.ds(r0, rows_per)],
                        acc_hbm.at[cid, pl.ds(r0, rows_per)])

    return sc_fn


def _pack_w_first(w, h_next):
    d_in, hc = w.shape
    return jnp.concatenate(
        [jnp.zeros((d_in, h_next), F32), w,
         jnp.zeros((d_in, ROWW - h_next - hc), F32)], axis=1)


def _pack_w_next(w, h_prev, h_next):
    wl = jnp.zeros((ROWW, ROWW), F32)
    return wl.at[h_prev:h_prev + w.shape[0],
                 h_next:h_next + w.shape[1]].set(w)


def _pack_avec(a, h_next):
    h, c = a.shape
    v = jnp.zeros((1, ROWW), F32)
    return v.at[0, h_next:h_next + h * c].set(a.reshape(-1))


def _repm(h, c):
    m = (_idxmap(h, c)[None, :] == np.arange(LANES)[:, None])
    return jnp.asarray(m.astype(np.float32))


def _bvec(h, b):
    v = jnp.zeros((1, ROWW), F32)
    return v.at[0, h:h + b.shape[0]].set(b)


def kernel(x, edge_index, W1, a_s1, a_d1, b1, W2, a_s2, a_d2, b2,
           W3, a_s3, a_d3, b3, Wl1, bl1, Wl2, bl2):
    n = x.shape[0]
    e = edge_index.shape[1]
    n_pad = ((n + 1 + _TCB - 1) // _TCB) * _TCB
    cw = -(-e // (NSC * NSUB * CHUNK))
    cw = ((cw + 7) // 8) * 8
    e_pad = NSC * NSUB * cw * CHUNK

    pad_e = e_pad - e
    src = jnp.concatenate(
        [edge_index[0], jnp.full((pad_e,), n, jnp.int32)]).reshape(-1, CHUNK)
    dst = jnp.concatenate(
        [edge_index[1], jnp.full((pad_e,), n, jnp.int32)]).reshape(-1, CHUNK)
    x_pad = jnp.pad(x, ((0, n_pad - n), (0, 0)))
    zer = jnp.zeros((n_pad, ROWW), F32)

    w1p = _pack_w_first(W1, 2)
    w2p = _pack_w_next(W2, 2, 2)
    w3p = _pack_w_next(W3, 2, 1)

    st1, dt1 = _proj(x_pad, w1p, _pack_avec(a_s1, 2), _pack_avec(a_d1, 2),
                     2, 5, n_pad)
    acc1 = _make_sc_edge(2, 5, n_pad, cw)(st1, dt1, src, dst, zer)
    st2, dt2 = _finish_proj(2, 10, 2, 2, acc1, st1, dt1, _repm(2, 5),
                            _bvec(2, b1), w2p,
                            _pack_avec(a_s2, 2), _pack_avec(a_d2, 2), n_pad)
    acc2 = _make_sc_edge(2, 2, n_pad, cw)(st2, dt2, src, dst, zer)
    st3, dt3 = _finish_proj(2, 4, 1, 2, acc2, st2, dt2, _repm(2, 2),
                            _bvec(2, b2), w3p,
                            _pack_avec(a_s3, 1), _pack_avec(a_d3, 1), n_pad)
    acc3 = _make_sc_edge(1, 2, n_pad, cw)(st3, dt3, src, dst, zer)
    hf3 = _finish_only(1, 2, acc3, st3, dt3, _repm(1, 2), _bvec(1, b3), n_pad)

    hh = hf3[:n, 1:3].reshape(n // 7, 14)
    out = _mlp(hh, Wl1, bl1, Wl2, bl2)
    return out[:, 0]


# speedup vs baseline: 1.2081x; 1.1911x over previous
import functools

import jax
import jax.numpy as jnp
import numpy as np
from jax import lax
from jax.experimental import pallas as pl
from jax.experimental.pallas import tpu as pltpu
from jax.experimental.pallas import tpu_sc as plsc

F32 = jnp.float32
NSC = 2
NSUB = 16
LANES = 16
CHUNK = 128
ROWW = 16
_TCB = 1792


def _idxmap(h, c):
    lane = np.arange(LANES)
    return np.where(lane < h, lane, np.clip((lane - h) // c, 0, h - 1)).astype(np.int32)


def _vdot(a, b):
    out = a[:, 0:1] * b[0:1, :]
    for i in range(1, a.shape[1]):
        out = out + a[:, i:i + 1] * b[i:i + 1, :]
    return out


def _alpha_heads(p, h_next, c_next):
    outs = []
    for hd in range(h_next):
        base = h_next + hd * c_next
        s = p[:, base:base + 1]
        for cc in range(1, c_next):
            s = s + p[:, base + cc:base + cc + 1]
        outs.append(s)
    return outs


def _proj_core(hm, avs_ref, avd_ref, h_next, c_next):
    lane = lax.broadcasted_iota(jnp.int32, hm.shape, 1)
    zerov = jnp.zeros_like(hm[:, 0:1])
    st = hm
    dt = jnp.zeros_like(hm)
    als = _alpha_heads(hm * avs_ref[...], h_next, c_next)
    ald = _alpha_heads(hm * avd_ref[...], h_next, c_next)
    for hd in range(h_next):
        st = st + jnp.where(lane == hd, als[hd], zerov)
        dt = dt + jnp.where(lane == hd, ald[hd], zerov)
    return st, dt


def _proj_body(h_next, c_next, x_ref, w_ref, avs_ref, avd_ref, st_ref, dt_ref):
    hm = jnp.dot(x_ref[...], w_ref[...], preferred_element_type=F32)
    st, dt = _proj_core(hm, avs_ref, avd_ref, h_next, c_next)
    st_ref[...] = st
    dt_ref[...] = dt


def _proj(x, w, avs, avd, h_next, c_next, n_pad):
    grid = n_pad // _TCB
    return pl.pallas_call(
        functools.partial(_proj_body, h_next, c_next),
        grid=(grid,),
        in_specs=[
            pl.BlockSpec((_TCB, x.shape[1]), lambda i: (i, 0)),
            pl.BlockSpec(w.shape, lambda i: (0, 0)),
            pl.BlockSpec((1, LANES), lambda i: (0, 0)),
            pl.BlockSpec((1, LANES), lambda i: (0, 0)),
        ],
        out_specs=[
            pl.BlockSpec((_TCB, ROWW), lambda i: (i, 0)),
            pl.BlockSpec((_TCB, ROWW), lambda i: (i, 0)),
        ],
        out_shape=[
            jax.ShapeDtypeStruct((n_pad, ROWW), F32),
            jax.ShapeDtypeStruct((n_pad, ROWW), F32),
        ],
    )(x, w, avs, avd)


def _finish_core(h, c, f, acc_ref, st_ref, dt_ref, bvec_ref):
    a = acc_ref[0] + acc_ref[1]
    s = st_ref[...]
    sl = s + dt_ref[...]
    ees = jnp.exp(jnp.maximum(sl, 0.2 * sl))
    lane = lax.broadcasted_iota(jnp.int32, s.shape, 1)
    if h == 1:
        rep = ees[:, 0:1]
    else:
        m0 = (lane == 0) | ((lane >= h) & (lane < h + c))
        rep = jnp.where(m0, ees[:, 0:1], ees[:, 1:2])
    tot = a + rep * jnp.where(lane < h, 1.0, s)
    if h == 1:
        den = tot[:, 0:1]
    else:
        den = jnp.where(m0, tot[:, 0:1], tot[:, 1:2])
    maskf = (lane >= h) & (lane < h + f)
    return jnp.maximum(jnp.where(maskf, tot / (den + 1e-16), 0.0)
                       + bvec_ref[...], 0.0)


def _finish_proj_body(h, c, f, h_next, c_next, acc_ref, st_ref, dt_ref,
                      bvec_ref, w_ref, avs_ref, avd_ref,
                      st2_ref, dt2_ref):
    hfeat = _finish_core(h, c, f, acc_ref, st_ref, dt_ref, bvec_ref)
    hm = jnp.dot(hfeat, w_ref[...], preferred_element_type=F32)
    st, dt = _proj_core(hm, avs_ref, avd_ref, h_next, c_next)
    st2_ref[...] = st
    dt2_ref[...] = dt


def _finish_proj(h, c, f, h_next, c_next, acc, st, dt, bvec, w, avs,
                 avd, n_pad):
    grid = n_pad // _TCB
    blk = lambda i: (i, 0)
    return pl.pallas_call(
        functools.partial(_finish_proj_body, h, c, f, h_next, c_next),
        grid=(grid,),
        in_specs=[
            pl.BlockSpec((NSC, _TCB, ROWW), lambda i: (0, i, 0)),
            pl.BlockSpec((_TCB, ROWW), blk),
            pl.BlockSpec((_TCB, ROWW), blk),
            pl.BlockSpec((1, LANES), lambda i: (0, 0)),
            pl.BlockSpec((LANES, LANES), lambda i: (0, 0)),
            pl.BlockSpec((1, LANES), lambda i: (0, 0)),
            pl.BlockSpec((1, LANES), lambda i: (0, 0)),
        ],
        out_specs=[
            pl.BlockSpec((_TCB, ROWW), blk),
            pl.BlockSpec((_TCB, ROWW), blk),
        ],
        out_shape=[
            jax.ShapeDtypeStruct((n_pad, ROWW), F32),
            jax.ShapeDtypeStruct((n_pad, ROWW), F32),
        ],
    )(acc, st, dt, bvec, w, avs, avd)


def _finish_only_body(h, c, f, acc_ref, st_ref, dt_ref, bvec_ref,
                      hf_ref):
    hf_ref[...] = _finish_core(h, c, f, acc_ref, st_ref, dt_ref, bvec_ref)


def _finish_only(h, c, f, acc, st, dt, bvec, n_pad):
    grid = n_pad // _TCB
    blk = lambda i: (i, 0)
    return pl.pallas_call(
        functools.partial(_finish_only_body, h, c, f),
        grid=(grid,),
        in_specs=[
            pl.BlockSpec((NSC, _TCB, ROWW), lambda i: (0, i, 0)),
            pl.BlockSpec((_TCB, ROWW), blk),
            pl.BlockSpec((_TCB, ROWW), blk),
            pl.BlockSpec((1, LANES), lambda i: (0, 0)),
        ],
        out_specs=pl.BlockSpec((_TCB, ROWW), blk),
        out_shape=jax.ShapeDtypeStruct((n_pad, ROWW), F32),
    )(acc, st, dt, bvec)


def _mlp_body(h_ref, w1_ref, b1_ref, w2_ref, b2_ref, o_ref):
    z = jnp.maximum(
        jnp.dot(h_ref[...], w1_ref[...], preferred_element_type=F32)
        + b1_ref[...], 0.0)
    o_ref[...] = (jnp.dot(z, w2_ref[...], preferred_element_type=F32)
                  + b2_ref[...])


def _mlp(hh, w1, b1, w2, b2):
    m = hh.shape[0]
    mb = 1000
    blk = lambda i: (i, 0)
    return pl.pallas_call(
        _mlp_body,
        grid=(m // mb,),
        in_specs=[
            pl.BlockSpec((mb, hh.shape[1]), blk),
            pl.BlockSpec(w1.shape, lambda i: (0, 0)),
            pl.BlockSpec((1, w1.shape[1]), lambda i: (0, 0)),
            pl.BlockSpec(w2.shape, lambda i: (0, 0)),
            pl.BlockSpec((1, 1), lambda i: (0, 0)),
        ],
        out_specs=pl.BlockSpec((mb, 1), blk),
        out_shape=jax.ShapeDtypeStruct((m, 1), F32),
    )(hh, w1, b1.reshape(1, -1), w2, b2.reshape(1, -1))


def _make_sc_edge(h, c, n_pad, cw):
    assert cw % 8 == 0
    rows_per = n_pad // NSUB
    ng8 = cw // 8
    mesh = plsc.VectorSubcoreMesh(core_axis_name="c", subcore_axis_name="s")

    scratch = (
        [pltpu.VMEM((CHUNK,), jnp.int32) for _ in range(8)]
        + [pltpu.VMEM((CHUNK,), jnp.int32) for _ in range(8)]
        + [pltpu.VMEM((CHUNK, ROWW), F32) for _ in range(4)]
        + [pltpu.VMEM((CHUNK, ROWW), F32) for _ in range(4)]
        + [pltpu.VMEM((CHUNK, ROWW), F32) for _ in range(4)]
        + [pltpu.VMEM_SHARED((n_pad, ROWW), F32)]
        + [pltpu.SemaphoreType.DMA for _ in range(16)]
    )

    @functools.partial(
        pl.kernel,
        mesh=mesh,
        compiler_params=pltpu.CompilerParams(use_tc_tiling_on_sc=False),
        out_type=jax.ShapeDtypeStruct((NSC, n_pad, ROWW), F32),
        scratch_types=scratch,
    )
    def sc_fn(st_hbm, dt_hbm, si_hbm, di_hbm, zer_hbm, acc_hbm, *scr):
        ixs = scr[0:8]
        ixc = scr[8:16]
        sb = scr[16:20]
        db = scr[20:24]
        ob = scr[24:28]
        acc_sp = scr[28]
        sem_i = scr[29:37]
        sem_g = scr[37:41]
        sem_s = scr[41:45]

        cid = lax.axis_index("c")
        sid = lax.axis_index("s")
        wid = cid * NSUB + sid
        r0 = sid * rows_per
        pltpu.sync_copy(zer_hbm.at[pl.ds(r0, rows_per)],
                        acc_sp.at[pl.ds(r0, rows_per)])
        plsc.subcore_barrier()

        def idx_issue(cc, r):
            row = wid * cw + cc
            pltpu.async_copy(si_hbm.at[row], ixs[r], sem_i[r])
            pltpu.async_copy(di_hbm.at[row], ixc[r], sem_i[r])

        def idx_wait(r):
            pltpu.make_async_copy(si_hbm.at[0], ixs[r], sem_i[r]).wait()
            pltpu.make_async_copy(di_hbm.at[0], ixc[r], sem_i[r]).wait()

        def g_issue(r, q):
            pltpu.async_copy(st_hbm.at[ixs[r]], sb[q], sem_g[q])
            pltpu.async_copy(dt_hbm.at[ixc[r]], db[q], sem_g[q])

        def g_wait(r, q):
            pltpu.make_async_copy(st_hbm.at[ixs[r]], sb[q], sem_g[q]).wait()
            pltpu.make_async_copy(dt_hbm.at[ixc[r]], db[q], sem_g[q]).wait()

        def s_issue(r, q):
            pltpu.async_copy(ob[q], acc_sp.at[ixc[r]], sem_s[q], add=True)

        def s_wait(r, q):
            pltpu.make_async_copy(ob[q], acc_sp.at[ixc[r]], sem_s[q]).wait()

        lane = lax.iota(jnp.int32, LANES)
        m0 = (lane == 0) | ((lane >= h) & (lane < h + c))
        mh = lane < h

        def compute(q):
            sbuf, dbuf, obuf = sb[q], db[q], ob[q]

            def edge(e, carry):
                s_row = sbuf[e]
                t = s_row + dbuf[e]
                ee = jnp.exp(jnp.maximum(t, 0.2 * t))
                if h == 1:
                    g = ee[0]
                else:
                    g = jnp.where(m0, ee[0], ee[1])
                obuf[e] = g * jnp.where(mh, 1.0, s_row)
                return carry

            lax.fori_loop(0, CHUNK, edge, 0, unroll=4)

        for r in range(8):
            idx_issue(r, r)
        idx_wait(0)
        g_issue(0, 0)
        idx_wait(1)
        g_issue(1, 1)

        def group(gg, carry):
            for qq in range(8):
                q = qq % 4
                cc = gg * 8 + qq
                g_wait(qq, q)
                if qq < 4:
                    @pl.when(gg >= 1)
                    def _():
                        s_wait((qq + 4) % 8, q)
                        idx_issue(cc + 4, (qq + 4) % 8)
                else:
                    s_wait(qq - 4, q)

                    @pl.when(gg < ng8 - 1)
                    def _():
                        idx_issue(cc + 4, qq - 4)
                if qq < 6:
                    idx_wait(qq + 2)
                    g_issue(qq + 2, (qq + 2) % 4)
                else:
                    @pl.when(gg < ng8 - 1)
                    def _():
                        idx_wait(qq - 6)
                        g_issue(qq - 6, (qq + 2) % 4)
                compute(q)
                s_issue(qq, q)
            return carry

        lax.fori_loop(0, ng8, group, 0)
        for qq in range(4, 8):
            s_wait(qq, qq % 4)
        plsc.subcore_barrier()
        pltpu.sync_copy(acc_sp.at[pl.ds(r0, rows_per)],
                        acc_hbm.at[cid, pl.ds(r0, rows_per)])

    return sc_fn


def _pack_w_first(w, h_next):
    d_in, hc = w.shape
    return jnp.concatenate(
        [jnp.zeros((d_in, h_next), F32), w,
         jnp.zeros((d_in, ROWW - h_next - hc), F32)], axis=1)


def _pack_w_next(w, h_prev, h_next):
    wl = jnp.zeros((ROWW, ROWW), F32)
    return wl.at[h_prev:h_prev + w.shape[0],
                 h_next:h_next + w.shape[1]].set(w)


def _pack_avec(a, h_next):
    h, c = a.shape
    v = jnp.zeros((1, ROWW), F32)
    return v.at[0, h_next:h_next + h * c].set(a.reshape(-1))


def _repm(h, c):
    m = (_idxmap(h, c)[None, :] == np.arange(LANES)[:, None])
    return jnp.asarray(m.astype(np.float32))


def _bvec(h, b):
    v = jnp.zeros((1, ROWW), F32)
    return v.at[0, h:h + b.shape[0]].set(b)


def kernel(x, edge_index, W1, a_s1, a_d1, b1, W2, a_s2, a_d2, b2,
           W3, a_s3, a_d3, b3, Wl1, bl1, Wl2, bl2):
    n = x.shape[0]
    e = edge_index.shape[1]
    n_pad = ((n + 1 + _TCB - 1) // _TCB) * _TCB
    cw = -(-e // (NSC * NSUB * CHUNK))
    cw = ((cw + 7) // 8) * 8
    e_pad = NSC * NSUB * cw * CHUNK

    pad_e = e_pad - e
    src = jnp.concatenate(
        [edge_index[0], jnp.full((pad_e,), n, jnp.int32)]).reshape(-1, CHUNK)
    dst = jnp.concatenate(
        [edge_index[1], jnp.full((pad_e,), n, jnp.int32)]).reshape(-1, CHUNK)
    x_pad = jnp.pad(x, ((0, n_pad - n), (0, 0)))
    zer = jnp.zeros((n_pad, ROWW), F32)

    w1p = _pack_w_first(W1, 2)
    w2p = _pack_w_next(W2, 2, 2)
    w3p = _pack_w_next(W3, 2, 1)

    st1, dt1 = _proj(x_pad, w1p, _pack_avec(a_s1, 2), _pack_avec(a_d1, 2),
                     2, 5, n_pad)
    acc1 = _make_sc_edge(2, 5, n_pad, cw)(st1, dt1, src, dst, zer)
    st2, dt2 = _finish_proj(2, 5, 10, 2, 2, acc1, st1, dt1,
                            _bvec(2, b1), w2p,
                            _pack_avec(a_s2, 2), _pack_avec(a_d2, 2), n_pad)
    acc2 = _make_sc_edge(2, 2, n_pad, cw)(st2, dt2, src, dst, zer)
    st3, dt3 = _finish_proj(2, 2, 4, 1, 2, acc2, st2, dt2,
                            _bvec(2, b2), w3p,
                            _pack_avec(a_s3, 1), _pack_avec(a_d3, 1), n_pad)
    acc3 = _make_sc_edge(1, 2, n_pad, cw)(st3, dt3, src, dst, zer)
    hf3 = _finish_only(1, 2, 2, acc3, st3, dt3, _bvec(1, b3), n_pad)

    hh = hf3[:n, 1:3].reshape(n // 7, 14)
    out = _mlp(hh, Wl1, bl1, Wl2, bl2)
    return out[:, 0]
